# baseline (speedup 1.0000x reference)
import jax
import jax.numpy as jnp
from jax.experimental import pallas as pl
from jax.experimental.pallas import tpu as pltpu


def _full_kernel(p_ref, w1_ref, b1_ref, w2_ref, b2_ref, w3_ref, b3_ref,
                 w4_ref, b4_ref, wf1_ref, bf1_ref, wf2_ref, bf2_ref,
                 wf3_ref, bf3_ref, o_ref):
    f32 = jnp.float32
    bb = p_ref.shape[1]

    a = p_ref[...].reshape(21, 21, bb, 64)
    g = jnp.concatenate(
        [a[ai:ai + 20, aj:aj + 20].reshape(400 * bb, 64)
         for ai in range(2) for aj in range(2)], axis=-1)
    acc = jnp.dot(g, w1_ref[...], preferred_element_type=f32)
    a1 = jnp.maximum(acc + b1_ref[...], 0.0).astype(jnp.bfloat16)

    a = a1.reshape(10, 2, 10, 2, bb, 32)
    g = jnp.concatenate(
        [a[ki // 2:ki // 2 + 9, ki % 2,
           kj // 2:kj // 2 + 9, kj % 2].reshape(81 * bb, 32)
         for ki in range(4) for kj in range(4)], axis=-1)
    acc = jnp.dot(g, w2_ref[...], preferred_element_type=f32)
    a2 = jnp.maximum(acc + b2_ref[...], 0.0).astype(jnp.bfloat16)
    a2 = a2.reshape(9, 9, bb, 64)

    acc = None
    for ki in range(3):
        g = jnp.concatenate(
            [a2[ki:ki + 7, kj:kj + 7].reshape(49 * bb, 64)
             for kj in range(3)], axis=-1)
        c = jnp.dot(g, w3_ref[ki], preferred_element_type=f32)
        acc = c if acc is None else acc + c
    a3 = jnp.maximum(acc + b3_ref[...], 0.0).astype(jnp.bfloat16)
    a3 = a3.reshape(7, 7, bb, 64)

    acc = None
    for ki in range(3):
        g = jnp.concatenate(
            [a3[ki:ki + 5, kj:kj + 5].reshape(25 * bb, 64)
             for kj in range(3)], axis=-1)
        c = jnp.dot(g, w4_ref[ki], preferred_element_type=f32)
        acc = c if acc is None else acc + c
    a4 = jnp.maximum(acc + b4_ref[...], 0.0).astype(jnp.bfloat16)
    a4 = a4.reshape(25, bb, 32)

    acc = None
    for p in range(5):
        g = jnp.concatenate([a4[5 * p + q] for q in range(5)], axis=-1)
        c = jnp.dot(g, wf1_ref[p], preferred_element_type=f32)
        acc = c if acc is None else acc + c
    h = jnp.maximum(acc + bf1_ref[...], 0.0).astype(jnp.bfloat16)
    h = jnp.dot(h, wf2_ref[...], preferred_element_type=f32) + bf2_ref[...]
    h = jnp.maximum(h, 0.0).astype(jnp.bfloat16)
    q = jnp.dot(h, wf3_ref[...], preferred_element_type=f32) + bf3_ref[...]
    o_ref[...] = q


def _const_specs(arrs):
    specs = []
    for a in arrs:
        nd = a.ndim
        specs.append(pl.BlockSpec(a.shape, lambda i, _nd=nd: (0,) * _nd))
    return specs


def _cast_kernel(x_ref, o_ref):
    o_ref[...] = x_ref[...].astype(jnp.bfloat16)


def _cast_bf16(x2):
    m = x2.shape[0]
    bm = 128 if m % 128 == 0 else m
    return pl.pallas_call(
        _cast_kernel,
        out_shape=jax.ShapeDtypeStruct(x2.shape, jnp.bfloat16),
        grid=(m // bm,),
        in_specs=[pl.BlockSpec((bm, x2.shape[1]), lambda i: (i, 0))],
        out_specs=pl.BlockSpec((bm, x2.shape[1]), lambda i: (i, 0)),
        compiler_params=pltpu.CompilerParams(
            dimension_semantics=("arbitrary",)),
    )(x2)


def _run_half(xh, consts):
    n = xh.shape[0]
    xb = _cast_bf16(xh.reshape(n * 4, 84 * 84))
    p0 = jnp.transpose(
        xb.reshape(n, 4, 21, 4, 21, 4),
        (2, 4, 0, 1, 3, 5)).reshape(441, n, 64)
    bb = 64 if n % 64 == 0 else n
    return pl.pallas_call(
        _full_kernel,
        out_shape=jax.ShapeDtypeStruct((n, 128), jnp.float32),
        grid=(n // bb,),
        in_specs=[pl.BlockSpec((441, bb, 64), lambda i: (0, i, 0))]
        + _const_specs(consts),
        out_specs=pl.BlockSpec((bb, 128), lambda i: (i, 0)),
        compiler_params=pltpu.CompilerParams(
            dimension_semantics=("arbitrary",)),
    )(p0, *consts)


def kernel(x, conv1_w, conv1_b, conv2_s, conv2_w, conv2_b,
           conv3_s, conv3_w, conv3_b, conv4_s, conv4_w, conv4_b,
           fc1_w, fc1_b, fc2_w, fc2_b, fc3_w, fc3_b):
    n = x.shape[0]
    w1 = jnp.transpose(conv1_w.reshape(2, 4, 2, 4, 4, 32),
                       (0, 2, 4, 1, 3, 5)).reshape(256, 32)
    consts = [w1, conv1_b,
              conv2_w.reshape(512, 64), conv2_b,
              conv3_w.reshape(3, 192, 64), conv3_b,
              conv4_w.reshape(3, 192, 32), conv4_b,
              fc1_w.reshape(5, 160, 512), fc1_b,
              fc2_w, fc2_b, fc3_w, fc3_b]
    if n % 128 == 0:
        q = jnp.concatenate([_run_half(x[:n // 2], consts),
                             _run_half(x[n // 2:], consts)], axis=0)
    else:
        q = _run_half(x, consts)
    return q[:, :6]


# speedup vs baseline: 8.8162x; 8.8162x over previous
import jax
import jax.numpy as jnp
from jax.experimental import pallas as pl
from jax.experimental.pallas import tpu as pltpu


def _full_kernel(p_ref, w1_ref, b1_ref, w2_ref, b2_ref, w3_ref, b3_ref,
                 w4_ref, b4_ref, wf1_ref, bf1_ref, wf2_ref, bf2_ref,
                 wf3_ref, bf3_ref, o_ref):
    f32 = jnp.float32
    bb = p_ref.shape[1]

    a = p_ref[...].reshape(21, 21, bb, 64)
    g = jnp.concatenate(
        [a[ai:ai + 20, aj:aj + 20].reshape(400 * bb, 64)
         for ai in range(2) for aj in range(2)], axis=-1)
    acc = jnp.dot(g, w1_ref[...], preferred_element_type=f32)
    a1 = jnp.maximum(acc + b1_ref[...], 0.0).astype(jnp.bfloat16)

    a = a1.reshape(10, 2, 10, 2, bb, 32)
    g = jnp.concatenate(
        [a[ki // 2:ki // 2 + 9, ki % 2,
           kj // 2:kj // 2 + 9, kj % 2].reshape(81 * bb, 32)
         for ki in range(4) for kj in range(4)], axis=-1)
    acc = jnp.dot(g, w2_ref[...], preferred_element_type=f32)
    a2 = jnp.maximum(acc + b2_ref[...], 0.0).astype(jnp.bfloat16)
    a2 = a2.reshape(9, 9, bb, 64)

    acc = None
    for ki in range(3):
        g = jnp.concatenate(
            [a2[ki:ki + 7, kj:kj + 7].reshape(49 * bb, 64)
             for kj in range(3)], axis=-1)
        c = jnp.dot(g, w3_ref[ki], preferred_element_type=f32)
        acc = c if acc is None else acc + c
    a3 = jnp.maximum(acc + b3_ref[...], 0.0).astype(jnp.bfloat16)
    a3 = a3.reshape(7, 7, bb, 64)

    acc = None
    for ki in range(3):
        g = jnp.concatenate(
            [a3[ki:ki + 5, kj:kj + 5].reshape(25 * bb, 64)
             for kj in range(3)], axis=-1)
        c = jnp.dot(g, w4_ref[ki], preferred_element_type=f32)
        acc = c if acc is None else acc + c
    a4 = jnp.maximum(acc + b4_ref[...], 0.0).astype(jnp.bfloat16)
    a4 = a4.reshape(25, bb, 32)

    acc = None
    for p in range(5):
        g = jnp.concatenate([a4[5 * p + q] for q in range(5)], axis=-1)
        c = jnp.dot(g, wf1_ref[p], preferred_element_type=f32)
        acc = c if acc is None else acc + c
    h = jnp.maximum(acc + bf1_ref[...], 0.0).astype(jnp.bfloat16)
    h = jnp.dot(h, wf2_ref[...], preferred_element_type=f32) + bf2_ref[...]
    h = jnp.maximum(h, 0.0).astype(jnp.bfloat16)
    q = jnp.dot(h, wf3_ref[...], preferred_element_type=f32) + bf3_ref[...]
    o_ref[...] = q


def _const_specs(arrs):
    specs = []
    for a in arrs:
        nd = a.ndim
        specs.append(pl.BlockSpec(a.shape, lambda i, _nd=nd: (0,) * _nd))
    return specs


def _run_half(xh, consts):
    n = xh.shape[0]
    p0 = jnp.transpose(
        xh.astype(jnp.bfloat16).reshape(n, 4, 21, 4, 21, 4),
        (2, 4, 0, 1, 3, 5)).reshape(441, n, 64)
    bb = 64 if n % 64 == 0 else n
    return pl.pallas_call(
        _full_kernel,
        out_shape=jax.ShapeDtypeStruct((n, 128), jnp.float32),
        grid=(n // bb,),
        in_specs=[pl.BlockSpec((441, bb, 64), lambda i: (0, i, 0))]
        + _const_specs(consts),
        out_specs=pl.BlockSpec((bb, 128), lambda i: (i, 0)),
        compiler_params=pltpu.CompilerParams(
            dimension_semantics=("arbitrary",)),
    )(p0, *consts)


def kernel(x, conv1_w, conv1_b, conv2_s, conv2_w, conv2_b,
           conv3_s, conv3_w, conv3_b, conv4_s, conv4_w, conv4_b,
           fc1_w, fc1_b, fc2_w, fc2_b, fc3_w, fc3_b):
    n = x.shape[0]
    w1 = jnp.transpose(conv1_w.reshape(2, 4, 2, 4, 4, 32),
                       (0, 2, 4, 1, 3, 5)).reshape(256, 32)
    consts = [w1, conv1_b,
              conv2_w.reshape(512, 64), conv2_b,
              conv3_w.reshape(3, 192, 64), conv3_b,
              conv4_w.reshape(3, 192, 32), conv4_b,
              fc1_w.reshape(5, 160, 512), fc1_b,
              fc2_w, fc2_b, fc3_w, fc3_b]
    if n % 128 == 0:
        q = jnp.concatenate([_run_half(x[:n // 2], consts),
                             _run_half(x[n // 2:], consts)], axis=0)
    else:
        q = _run_half(x, consts)
    return q[:, :6]


# speedup vs baseline: 8.9679x; 1.0172x over previous
import jax
import jax.numpy as jnp
from jax.experimental import pallas as pl
from jax.experimental.pallas import tpu as pltpu


def _full_kernel(p_ref, w1_ref, b1_ref, w2_ref, b2_ref, w3_ref, b3_ref,
                 w4_ref, b4_ref, wf1_ref, bf1_ref, wf2_ref, bf2_ref,
                 wf3_ref, bf3_ref, o_ref):
    f32 = jnp.float32
    bb = p_ref.shape[1]

    a = p_ref[...].reshape(21, 21, bb, 64)
    g = jnp.concatenate(
        [a[ai:ai + 20, aj:aj + 20].reshape(400 * bb, 64)
         for ai in range(2) for aj in range(2)], axis=-1)
    acc = jnp.dot(g, w1_ref[...], preferred_element_type=f32)
    a1 = jnp.maximum(acc + b1_ref[...], 0.0).astype(jnp.bfloat16)

    a = a1.reshape(10, 2, 10, 2, bb, 32)
    g = jnp.concatenate(
        [a[ki // 2:ki // 2 + 9, ki % 2,
           kj // 2:kj // 2 + 9, kj % 2].reshape(81 * bb, 32)
         for ki in range(4) for kj in range(4)], axis=-1)
    acc = jnp.dot(g, w2_ref[...], preferred_element_type=f32)
    a2 = jnp.maximum(acc + b2_ref[...], 0.0).astype(jnp.bfloat16)
    a2 = a2.reshape(9, 9, bb, 64)

    g = jnp.concatenate(
        [a2[ki:ki + 7, kj:kj + 7].reshape(49 * bb, 64)
         for ki in range(3) for kj in range(3)], axis=-1)
    acc = jnp.dot(g, w3_ref[...], preferred_element_type=f32)
    a3 = jnp.maximum(acc + b3_ref[...], 0.0).astype(jnp.bfloat16)
    a3 = a3.reshape(7, 7, bb, 64)

    g = jnp.concatenate(
        [a3[ki:ki + 5, kj:kj + 5].reshape(25 * bb, 64)
         for ki in range(3) for kj in range(3)], axis=-1)
    acc = jnp.dot(g, w4_ref[...], preferred_element_type=f32)
    a4 = jnp.maximum(acc + b4_ref[...], 0.0).astype(jnp.bfloat16)
    a4 = a4.reshape(25, bb, 32)

    g = jnp.concatenate([a4[p] for p in range(25)], axis=-1)
    acc = jnp.dot(g, wf1_ref[...], preferred_element_type=f32)
    h = jnp.maximum(acc + bf1_ref[...], 0.0).astype(jnp.bfloat16)
    h = jnp.dot(h, wf2_ref[...], preferred_element_type=f32) + bf2_ref[...]
    h = jnp.maximum(h, 0.0).astype(jnp.bfloat16)
    q = jnp.dot(h, wf3_ref[...], preferred_element_type=f32) + bf3_ref[...]
    o_ref[...] = q


def _const_specs(arrs):
    specs = []
    for a in arrs:
        nd = a.ndim
        specs.append(pl.BlockSpec(a.shape, lambda i, _nd=nd: (0,) * _nd))
    return specs


def _run_half(xh, consts):
    n = xh.shape[0]
    p0 = jnp.transpose(
        xh.astype(jnp.bfloat16).reshape(n, 4, 21, 4, 21, 4),
        (2, 4, 0, 1, 3, 5)).reshape(441, n, 64)
    bb = 64 if n % 64 == 0 else n
    return pl.pallas_call(
        _full_kernel,
        out_shape=jax.ShapeDtypeStruct((n, 128), jnp.float32),
        grid=(n // bb,),
        in_specs=[pl.BlockSpec((441, bb, 64), lambda i: (0, i, 0))]
        + _const_specs(consts),
        out_specs=pl.BlockSpec((bb, 128), lambda i: (i, 0)),
        compiler_params=pltpu.CompilerParams(
            dimension_semantics=("arbitrary",)),
    )(p0, *consts)


def kernel(x, conv1_w, conv1_b, conv2_s, conv2_w, conv2_b,
           conv3_s, conv3_w, conv3_b, conv4_s, conv4_w, conv4_b,
           fc1_w, fc1_b, fc2_w, fc2_b, fc3_w, fc3_b):
    n = x.shape[0]
    w1 = jnp.transpose(conv1_w.reshape(2, 4, 2, 4, 4, 32),
                       (0, 2, 4, 1, 3, 5)).reshape(256, 32)
    consts = [w1, conv1_b,
              conv2_w.reshape(512, 64), conv2_b,
              conv3_w.reshape(576, 64), conv3_b,
              conv4_w.reshape(576, 32), conv4_b,
              fc1_w.reshape(800, 512), fc1_b,
              fc2_w, fc2_b, fc3_w, fc3_b]
    if n % 128 == 0:
        q = jnp.concatenate([_run_half(x[:n // 2], consts),
                             _run_half(x[n // 2:], consts)], axis=0)
    else:
        q = _run_half(x, consts)
    return q[:, :6]
